# baseline (speedup 1.0000x reference)
import functools

import jax
import jax.numpy as jnp
from jax import lax
from jax.experimental import pallas as pl
from jax.experimental.pallas import tpu as pltpu
from jax.experimental.pallas import tpu_sc as plsc

N_NODES = 10000
D = 128
NC = 2
NS = 16
LANES = 16
NW = NC * NS
CHUNK = 96
N_ACC = 10240
ROW_B = 1000
CORE0_FRAC = 0.32


def _sc_mesh():
    return plsc.VectorSubcoreMesh(
        core_axis_name="c", subcore_axis_name="s",
        num_cores=NC, num_subcores=NS)


def _make_deg_kernel(n_chunks):
    @functools.partial(
        pl.kernel,
        out_type=jax.ShapeDtypeStruct((NC, N_ACC, D), jnp.float32),
        mesh=_sc_mesh(),
        scratch_types=[
            pltpu.VMEM((n_chunks, CHUNK), jnp.int32),
            pltpu.VMEM((CHUNK, D), jnp.float32),
            pltpu.VMEM_SHARED((N_ACC, D), jnp.float32),
        ],
    )
    def deg_kernel(dst_hbm, zz_hbm, out_hbm, dst_v, ones_v, acc_sh):
        c = lax.axis_index("c")
        s = lax.axis_index("s")
        w = c * NS + s
        pltpu.sync_copy(dst_hbm.at[w], dst_v)

        rows_per_tile = N_ACC // NS
        onesvec = jnp.ones((LANES,), jnp.float32)
        per_row = D // LANES

        def fill(i, _):
            r = i // per_row
            k = i - r * per_row
            ones_v[r, pl.ds(k * LANES, LANES)] = onesvec
            return 0
        lax.fori_loop(0, CHUNK * per_row, fill, 0)

        pltpu.sync_copy(
            zz_hbm, acc_sh.at[pl.ds(s * rows_per_tile, rows_per_tile)])
        plsc.subcore_barrier()

        def body(j, _):
            pltpu.sync_copy(ones_v, acc_sh.at[dst_v.at[j]], add=True)
            return 0
        lax.fori_loop(0, n_chunks, body, 0)
        plsc.subcore_barrier()

        pltpu.sync_copy(acc_sh.at[pl.ds(s * rows_per_tile, rows_per_tile)],
                        out_hbm.at[c, pl.ds(s * rows_per_tile, rows_per_tile)])

    return deg_kernel


def _make_scatter_kernel(k0, k1):
    @functools.partial(
        pl.kernel,
        out_type=jax.ShapeDtypeStruct((NC, N_ACC, D), jnp.float32),
        mesh=_sc_mesh(),
        scratch_types=[
            pltpu.VMEM((2, CHUNK), jnp.int32),
            pltpu.VMEM((2, CHUNK), jnp.int32),
            pltpu.VMEM((2, CHUNK), jnp.int32),
            pltpu.VMEM((CHUNK, D), jnp.float32),
            pltpu.VMEM((CHUNK, D), jnp.float32),
            pltpu.VMEM((CHUNK, D), jnp.float32),
            pltpu.VMEM_SHARED((N_ACC, D), jnp.float32),
            pltpu.SemaphoreType.DMA,
            pltpu.SemaphoreType.DMA,
            pltpu.SemaphoreType.DMA,
            pltpu.SemaphoreType.DMA,
            pltpu.SemaphoreType.DMA,
            pltpu.SemaphoreType.DMA,
        ],
    )
    def scat_kernel(g_hbm, ei_hbm, zz_hbm, out_hbm,
                    ia, ib, ic, bufa, bufb, bufc, acc_sh,
                    sema, semb, semc, semia, semib, semic):
        c = lax.axis_index("c")
        s = lax.axis_index("s")
        base = jnp.where(c == 0, s * k0, NS * k0 + s * k1)
        n_chunks = jnp.where(c == 0, k0, k1)
        rows_per_tile = N_ACC // NS

        pltpu.sync_copy(
            zz_hbm, acc_sh.at[pl.ds(s * rows_per_tile, rows_per_tile)])
        plsc.subcore_barrier()

        pltpu.sync_copy(ei_hbm.at[base], ia)
        pltpu.sync_copy(ei_hbm.at[base + 1], ib)
        pltpu.sync_copy(ei_hbm.at[base + 2], ic)
        pltpu.async_copy(g_hbm.at[ia.at[0]], bufa, sema)
        pltpu.async_copy(g_hbm.at[ib.at[0]], bufb, semb)
        pltpu.async_copy(g_hbm.at[ic.at[0]], bufc, semc)

        def body(i, _):
            j = 3 * i
            pltpu.make_async_copy(g_hbm.at[ia.at[0]], bufa, sema).wait()
            pltpu.sync_copy(bufa, acc_sh.at[ia.at[1]], add=True)

            @pl.when(j + 3 < n_chunks)
            def _():
                pltpu.async_copy(ei_hbm.at[base + j + 3], ia, semia)

            @pl.when(j + 1 < n_chunks)
            def _():
                pltpu.make_async_copy(g_hbm.at[ib.at[0]], bufb, semb).wait()
                pltpu.sync_copy(bufb, acc_sh.at[ib.at[1]], add=True)

            @pl.when(j + 3 < n_chunks)
            def _():
                pltpu.make_async_copy(ei_hbm.at[base + j + 3], ia, semia).wait()
                pltpu.async_copy(g_hbm.at[ia.at[0]], bufa, sema)

            @pl.when(j + 4 < n_chunks)
            def _():
                pltpu.async_copy(ei_hbm.at[base + j + 4], ib, semib)

            @pl.when(j + 2 < n_chunks)
            def _():
                pltpu.make_async_copy(g_hbm.at[ic.at[0]], bufc, semc).wait()
                pltpu.sync_copy(bufc, acc_sh.at[ic.at[1]], add=True)

            @pl.when(j + 4 < n_chunks)
            def _():
                pltpu.make_async_copy(ei_hbm.at[base + j + 4], ib, semib).wait()
                pltpu.async_copy(g_hbm.at[ib.at[0]], bufb, semb)

            @pl.when(j + 5 < n_chunks)
            def _():
                pltpu.async_copy(ei_hbm.at[base + j + 5], ic, semic)
                pltpu.make_async_copy(ei_hbm.at[base + j + 5], ic, semic).wait()
                pltpu.async_copy(g_hbm.at[ic.at[0]], bufc, semc)
            return 0
        lax.fori_loop(0, (n_chunks + 2) // 3, body, 0)
        plsc.subcore_barrier()

        pltpu.sync_copy(acc_sh.at[pl.ds(s * rows_per_tile, rows_per_tile)],
                        out_hbm.at[c, pl.ds(s * rows_per_tile, rows_per_tile)])

    return scat_kernel


def _tc_scale_matmul(x, W, dinv):
    def body(x_ref, w_ref, d_ref, o_ref):
        h = jnp.dot(x_ref[...], w_ref[...], preferred_element_type=jnp.float32)
        o_ref[...] = h * d_ref[...]

    return pl.pallas_call(
        body,
        grid=(N_NODES // ROW_B,),
        in_specs=[
            pl.BlockSpec((ROW_B, D), lambda i: (i, 0)),
            pl.BlockSpec((D, D), lambda i: (0, 0)),
            pl.BlockSpec((ROW_B, 1), lambda i: (i, 0)),
        ],
        out_specs=pl.BlockSpec((ROW_B, D), lambda i: (i, 0)),
        out_shape=jax.ShapeDtypeStruct((N_NODES, D), jnp.float32),
    )(x, W, dinv)


def _tc_mid(parts, g, dinv, b, W):
    def body(p_ref, g_ref, d_ref, b_ref, w_ref, o_ref):
        agg = p_ref[0] + p_ref[1] + g_ref[...]
        t = jnp.maximum(agg * d_ref[...] + b_ref[...], 0.0)
        h = jnp.dot(t, w_ref[...], preferred_element_type=jnp.float32)
        o_ref[...] = h * d_ref[...]

    return pl.pallas_call(
        body,
        grid=(N_NODES // ROW_B,),
        in_specs=[
            pl.BlockSpec((NC, ROW_B, D), lambda i: (0, i, 0)),
            pl.BlockSpec((ROW_B, D), lambda i: (i, 0)),
            pl.BlockSpec((ROW_B, 1), lambda i: (i, 0)),
            pl.BlockSpec((1, D), lambda i: (0, 0)),
            pl.BlockSpec((D, D), lambda i: (0, 0)),
        ],
        out_specs=pl.BlockSpec((ROW_B, D), lambda i: (i, 0)),
        out_shape=jax.ShapeDtypeStruct((N_NODES, D), jnp.float32),
    )(parts, g, dinv, b, W)


def _tc_out(parts, g, dinv, b):
    def body(p_ref, g_ref, d_ref, b_ref, o_ref):
        agg = p_ref[0] + p_ref[1] + g_ref[...]
        z = agg * d_ref[...] + b_ref[...]
        m = jnp.max(z, axis=1, keepdims=True)
        e = jnp.exp(z - m)
        ssum = jnp.sum(e, axis=1, keepdims=True)
        o_ref[...] = (z - m) - jnp.log(ssum)

    return pl.pallas_call(
        body,
        grid=(N_NODES // ROW_B,),
        in_specs=[
            pl.BlockSpec((NC, ROW_B, D), lambda i: (0, i, 0)),
            pl.BlockSpec((ROW_B, D), lambda i: (i, 0)),
            pl.BlockSpec((ROW_B, 1), lambda i: (i, 0)),
            pl.BlockSpec((1, D), lambda i: (0, 0)),
        ],
        out_specs=pl.BlockSpec((ROW_B, D), lambda i: (i, 0)),
        out_shape=jax.ShapeDtypeStruct((N_NODES, D), jnp.float32),
    )(parts, g, dinv, b)


def kernel(x, edge_index, W1, b1, W2, b2):
    src = edge_index[0]
    dst = edge_index[1]
    e = src.shape[0]
    grain = NW * CHUNK
    e_pad = ((e + grain - 1) // grain) * grain
    n_chunks = e_pad // grain

    src_p = jnp.concatenate(
        [src, jnp.zeros((e_pad - e,), jnp.int32)]).reshape(NW, n_chunks, CHUNK)
    dst_p = jnp.concatenate(
        [dst, jnp.full((e_pad - e,), N_NODES, jnp.int32)]
    ).reshape(NW, n_chunks, CHUNK)
    t_chunks = e_pad // CHUNK
    ei_flat = jnp.stack(
        [src_p.reshape(t_chunks, CHUNK), dst_p.reshape(t_chunks, CHUNK)],
        axis=1)
    zz = jnp.zeros((N_ACC // NS, D), jnp.float32)

    deg_parts = _make_deg_kernel(n_chunks)(dst_p, zz)
    deg = deg_parts[:, :N_NODES, 0].sum(axis=0) + 1.0
    dinv = lax.rsqrt(deg).reshape(N_NODES, 1)

    k0 = int(round(CORE0_FRAC * t_chunks / NS))
    k1 = t_chunks // NS - k0
    scat = _make_scatter_kernel(k0, k1)
    b1r = b1.reshape(1, D)
    b2r = b2.reshape(1, D)

    g1 = _tc_scale_matmul(x, W1, dinv)
    parts1 = scat(g1, ei_flat, zz)[:, :N_NODES, :]
    g2 = _tc_mid(parts1, g1, dinv, b1r, W2)
    parts2 = scat(g2, ei_flat, zz)[:, :N_NODES, :]
    return _tc_out(parts2, g2, dinv, b2r)


# speedup vs baseline: 1.1679x; 1.1679x over previous
import functools

import jax
import jax.numpy as jnp
from jax import lax
from jax.experimental import pallas as pl
from jax.experimental.pallas import tpu as pltpu
from jax.experimental.pallas import tpu_sc as plsc

N_NODES = 10000
D = 128
NC = 2
NS = 16
LANES = 16
NW = NC * NS
CHUNK = 96
N_ACC = 10240
ROW_B = 1000
CORE0_FRAC = 0.68


def _sc_mesh():
    return plsc.VectorSubcoreMesh(
        core_axis_name="c", subcore_axis_name="s",
        num_cores=NC, num_subcores=NS)


def _make_deg_kernel(n_chunks):
    @functools.partial(
        pl.kernel,
        out_type=jax.ShapeDtypeStruct((NC, N_ACC, D), jnp.float32),
        mesh=_sc_mesh(),
        scratch_types=[
            pltpu.VMEM((n_chunks, CHUNK), jnp.int32),
            pltpu.VMEM((CHUNK, D), jnp.float32),
            pltpu.VMEM_SHARED((N_ACC, D), jnp.float32),
        ],
    )
    def deg_kernel(dst_hbm, zz_hbm, out_hbm, dst_v, ones_v, acc_sh):
        c = lax.axis_index("c")
        s = lax.axis_index("s")
        w = c * NS + s
        pltpu.sync_copy(dst_hbm.at[w], dst_v)

        rows_per_tile = N_ACC // NS
        onesvec = jnp.ones((LANES,), jnp.float32)
        per_row = D // LANES

        def fill(i, _):
            r = i // per_row
            k = i - r * per_row
            ones_v[r, pl.ds(k * LANES, LANES)] = onesvec
            return 0
        lax.fori_loop(0, CHUNK * per_row, fill, 0)

        pltpu.sync_copy(
            zz_hbm, acc_sh.at[pl.ds(s * rows_per_tile, rows_per_tile)])
        plsc.subcore_barrier()

        def body(j, _):
            pltpu.sync_copy(ones_v, acc_sh.at[dst_v.at[j]], add=True)
            return 0
        lax.fori_loop(0, n_chunks, body, 0)
        plsc.subcore_barrier()

        pltpu.sync_copy(acc_sh.at[pl.ds(s * rows_per_tile, rows_per_tile)],
                        out_hbm.at[c, pl.ds(s * rows_per_tile, rows_per_tile)])

    return deg_kernel


def _make_scatter_kernel(k0, k1):
    @functools.partial(
        pl.kernel,
        out_type=jax.ShapeDtypeStruct((NC, N_ACC, D), jnp.float32),
        mesh=_sc_mesh(),
        scratch_types=[
            pltpu.VMEM((2, CHUNK), jnp.int32),
            pltpu.VMEM((2, CHUNK), jnp.int32),
            pltpu.VMEM((2, CHUNK), jnp.int32),
            pltpu.VMEM((CHUNK, D), jnp.float32),
            pltpu.VMEM((CHUNK, D), jnp.float32),
            pltpu.VMEM((CHUNK, D), jnp.float32),
            pltpu.VMEM_SHARED((N_ACC, D), jnp.float32),
            pltpu.SemaphoreType.DMA,
            pltpu.SemaphoreType.DMA,
            pltpu.SemaphoreType.DMA,
            pltpu.SemaphoreType.DMA,
            pltpu.SemaphoreType.DMA,
            pltpu.SemaphoreType.DMA,
        ],
    )
    def scat_kernel(g_hbm, ei_hbm, zz_hbm, out_hbm,
                    ia, ib, ic, bufa, bufb, bufc, acc_sh,
                    sema, semb, semc, semia, semib, semic):
        c = lax.axis_index("c")
        s = lax.axis_index("s")
        base = jnp.where(c == 0, s * k0, NS * k0 + s * k1)
        n_chunks = jnp.where(c == 0, k0, k1)
        rows_per_tile = N_ACC // NS

        pltpu.sync_copy(
            zz_hbm, acc_sh.at[pl.ds(s * rows_per_tile, rows_per_tile)])
        plsc.subcore_barrier()

        pltpu.sync_copy(ei_hbm.at[base], ia)
        pltpu.sync_copy(ei_hbm.at[base + 1], ib)
        pltpu.sync_copy(ei_hbm.at[base + 2], ic)
        pltpu.async_copy(g_hbm.at[ia.at[0]], bufa, sema)
        pltpu.async_copy(g_hbm.at[ib.at[0]], bufb, semb)
        pltpu.async_copy(g_hbm.at[ic.at[0]], bufc, semc)

        def body(i, _):
            j = 3 * i
            pltpu.make_async_copy(g_hbm.at[ia.at[0]], bufa, sema).wait()
            pltpu.sync_copy(bufa, acc_sh.at[ia.at[1]], add=True)

            @pl.when(j + 3 < n_chunks)
            def _():
                pltpu.async_copy(ei_hbm.at[base + j + 3], ia, semia)

            @pl.when(j + 1 < n_chunks)
            def _():
                pltpu.make_async_copy(g_hbm.at[ib.at[0]], bufb, semb).wait()
                pltpu.sync_copy(bufb, acc_sh.at[ib.at[1]], add=True)

            @pl.when(j + 3 < n_chunks)
            def _():
                pltpu.make_async_copy(ei_hbm.at[base + j + 3], ia, semia).wait()
                pltpu.async_copy(g_hbm.at[ia.at[0]], bufa, sema)

            @pl.when(j + 4 < n_chunks)
            def _():
                pltpu.async_copy(ei_hbm.at[base + j + 4], ib, semib)

            @pl.when(j + 2 < n_chunks)
            def _():
                pltpu.make_async_copy(g_hbm.at[ic.at[0]], bufc, semc).wait()
                pltpu.sync_copy(bufc, acc_sh.at[ic.at[1]], add=True)

            @pl.when(j + 4 < n_chunks)
            def _():
                pltpu.make_async_copy(ei_hbm.at[base + j + 4], ib, semib).wait()
                pltpu.async_copy(g_hbm.at[ib.at[0]], bufb, semb)

            @pl.when(j + 5 < n_chunks)
            def _():
                pltpu.async_copy(ei_hbm.at[base + j + 5], ic, semic)
                pltpu.make_async_copy(ei_hbm.at[base + j + 5], ic, semic).wait()
                pltpu.async_copy(g_hbm.at[ic.at[0]], bufc, semc)
            return 0
        lax.fori_loop(0, (n_chunks + 2) // 3, body, 0)
        plsc.subcore_barrier()

        pltpu.sync_copy(acc_sh.at[pl.ds(s * rows_per_tile, rows_per_tile)],
                        out_hbm.at[c, pl.ds(s * rows_per_tile, rows_per_tile)])

    return scat_kernel


def _tc_scale_matmul(x, W, dinv):
    def body(x_ref, w_ref, d_ref, o_ref):
        h = jnp.dot(x_ref[...], w_ref[...], preferred_element_type=jnp.float32)
        o_ref[...] = h * d_ref[...]

    return pl.pallas_call(
        body,
        grid=(N_NODES // ROW_B,),
        in_specs=[
            pl.BlockSpec((ROW_B, D), lambda i: (i, 0)),
            pl.BlockSpec((D, D), lambda i: (0, 0)),
            pl.BlockSpec((ROW_B, 1), lambda i: (i, 0)),
        ],
        out_specs=pl.BlockSpec((ROW_B, D), lambda i: (i, 0)),
        out_shape=jax.ShapeDtypeStruct((N_NODES, D), jnp.float32),
    )(x, W, dinv)


def _tc_mid(parts, g, dinv, b, W):
    def body(p_ref, g_ref, d_ref, b_ref, w_ref, o_ref):
        agg = p_ref[0] + p_ref[1] + g_ref[...]
        t = jnp.maximum(agg * d_ref[...] + b_ref[...], 0.0)
        h = jnp.dot(t, w_ref[...], preferred_element_type=jnp.float32)
        o_ref[...] = h * d_ref[...]

    return pl.pallas_call(
        body,
        grid=(N_NODES // ROW_B,),
        in_specs=[
            pl.BlockSpec((NC, ROW_B, D), lambda i: (0, i, 0)),
            pl.BlockSpec((ROW_B, D), lambda i: (i, 0)),
            pl.BlockSpec((ROW_B, 1), lambda i: (i, 0)),
            pl.BlockSpec((1, D), lambda i: (0, 0)),
            pl.BlockSpec((D, D), lambda i: (0, 0)),
        ],
        out_specs=pl.BlockSpec((ROW_B, D), lambda i: (i, 0)),
        out_shape=jax.ShapeDtypeStruct((N_NODES, D), jnp.float32),
    )(parts, g, dinv, b, W)


def _tc_out(parts, g, dinv, b):
    def body(p_ref, g_ref, d_ref, b_ref, o_ref):
        agg = p_ref[0] + p_ref[1] + g_ref[...]
        z = agg * d_ref[...] + b_ref[...]
        m = jnp.max(z, axis=1, keepdims=True)
        e = jnp.exp(z - m)
        ssum = jnp.sum(e, axis=1, keepdims=True)
        o_ref[...] = (z - m) - jnp.log(ssum)

    return pl.pallas_call(
        body,
        grid=(N_NODES // ROW_B,),
        in_specs=[
            pl.BlockSpec((NC, ROW_B, D), lambda i: (0, i, 0)),
            pl.BlockSpec((ROW_B, D), lambda i: (i, 0)),
            pl.BlockSpec((ROW_B, 1), lambda i: (i, 0)),
            pl.BlockSpec((1, D), lambda i: (0, 0)),
        ],
        out_specs=pl.BlockSpec((ROW_B, D), lambda i: (i, 0)),
        out_shape=jax.ShapeDtypeStruct((N_NODES, D), jnp.float32),
    )(parts, g, dinv, b)


def kernel(x, edge_index, W1, b1, W2, b2):
    src = edge_index[0]
    dst = edge_index[1]
    e = src.shape[0]
    grain = NW * CHUNK
    e_pad = ((e + grain - 1) // grain) * grain
    n_chunks = e_pad // grain

    src_p = jnp.concatenate(
        [src, jnp.zeros((e_pad - e,), jnp.int32)]).reshape(NW, n_chunks, CHUNK)
    dst_p = jnp.concatenate(
        [dst, jnp.full((e_pad - e,), N_NODES, jnp.int32)]
    ).reshape(NW, n_chunks, CHUNK)
    t_chunks = e_pad // CHUNK
    ei_flat = jnp.stack(
        [src_p.reshape(t_chunks, CHUNK), dst_p.reshape(t_chunks, CHUNK)],
        axis=1)
    zz = jnp.zeros((N_ACC // NS, D), jnp.float32)

    deg_parts = _make_deg_kernel(n_chunks)(dst_p, zz)
    deg = deg_parts[:, :N_NODES, 0].sum(axis=0) + 1.0
    dinv = lax.rsqrt(deg).reshape(N_NODES, 1)

    k0 = int(round(CORE0_FRAC * t_chunks / NS))
    k1 = t_chunks // NS - k0
    scat = _make_scatter_kernel(k0, k1)
    b1r = b1.reshape(1, D)
    b2r = b2.reshape(1, D)

    g1 = _tc_scale_matmul(x, W1, dinv)
    parts1 = scat(g1, ei_flat, zz)[:, :N_NODES, :]
    g2 = _tc_mid(parts1, g1, dinv, b1r, W2)
    parts2 = scat(g2, ei_flat, zz)[:, :N_NODES, :]
    return _tc_out(parts2, g2, dinv, b2r)


# speedup vs baseline: 1.2109x; 1.0369x over previous
import functools

import jax
import jax.numpy as jnp
from jax import lax
from jax.experimental import pallas as pl
from jax.experimental.pallas import tpu as pltpu
from jax.experimental.pallas import tpu_sc as plsc

N_NODES = 10000
D = 128
NC = 2
NS = 16
LANES = 16
NW = NC * NS
CHUNK = 96
N_ACC = 10240
ROW_B = 1000
CORE0_FRAC = 0.75


def _sc_mesh():
    return plsc.VectorSubcoreMesh(
        core_axis_name="c", subcore_axis_name="s",
        num_cores=NC, num_subcores=NS)


def _make_deg_kernel(n_chunks):
    @functools.partial(
        pl.kernel,
        out_type=jax.ShapeDtypeStruct((NC, N_ACC, D), jnp.float32),
        mesh=_sc_mesh(),
        scratch_types=[
            pltpu.VMEM((n_chunks, CHUNK), jnp.int32),
            pltpu.VMEM((CHUNK, D), jnp.float32),
            pltpu.VMEM_SHARED((N_ACC, D), jnp.float32),
        ],
    )
    def deg_kernel(dst_hbm, zz_hbm, out_hbm, dst_v, ones_v, acc_sh):
        c = lax.axis_index("c")
        s = lax.axis_index("s")
        w = c * NS + s
        pltpu.sync_copy(dst_hbm.at[w], dst_v)

        rows_per_tile = N_ACC // NS
        onesvec = jnp.ones((LANES,), jnp.float32)
        per_row = D // LANES

        def fill(i, _):
            r = i // per_row
            k = i - r * per_row
            ones_v[r, pl.ds(k * LANES, LANES)] = onesvec
            return 0
        lax.fori_loop(0, CHUNK * per_row, fill, 0)

        pltpu.sync_copy(
            zz_hbm, acc_sh.at[pl.ds(s * rows_per_tile, rows_per_tile)])
        plsc.subcore_barrier()

        def body(j, _):
            pltpu.sync_copy(ones_v, acc_sh.at[dst_v.at[j]], add=True)
            return 0
        lax.fori_loop(0, n_chunks, body, 0)
        plsc.subcore_barrier()

        pltpu.sync_copy(acc_sh.at[pl.ds(s * rows_per_tile, rows_per_tile)],
                        out_hbm.at[c, pl.ds(s * rows_per_tile, rows_per_tile)])

    return deg_kernel


def _make_scatter_kernel(k0, k1):
    @functools.partial(
        pl.kernel,
        out_type=jax.ShapeDtypeStruct((NC, N_ACC, D), jnp.float32),
        mesh=_sc_mesh(),
        scratch_types=[
            pltpu.VMEM((2, CHUNK), jnp.int32),
            pltpu.VMEM((2, CHUNK), jnp.int32),
            pltpu.VMEM((2, CHUNK), jnp.int32),
            pltpu.VMEM((CHUNK, D), jnp.float32),
            pltpu.VMEM((CHUNK, D), jnp.float32),
            pltpu.VMEM((CHUNK, D), jnp.float32),
            pltpu.VMEM_SHARED((N_ACC, D), jnp.float32),
            pltpu.SemaphoreType.DMA,
            pltpu.SemaphoreType.DMA,
            pltpu.SemaphoreType.DMA,
            pltpu.SemaphoreType.DMA,
            pltpu.SemaphoreType.DMA,
            pltpu.SemaphoreType.DMA,
        ],
    )
    def scat_kernel(g_hbm, ei_hbm, zz_hbm, out_hbm,
                    ia, ib, ic, bufa, bufb, bufc, acc_sh,
                    sema, semb, semc, semia, semib, semic):
        c = lax.axis_index("c")
        s = lax.axis_index("s")
        base = jnp.where(c == 0, s * k0, NS * k0 + s * k1)
        n_chunks = jnp.where(c == 0, k0, k1)
        rows_per_tile = N_ACC // NS

        pltpu.sync_copy(
            zz_hbm, acc_sh.at[pl.ds(s * rows_per_tile, rows_per_tile)])
        plsc.subcore_barrier()

        pltpu.sync_copy(ei_hbm.at[base], ia)
        pltpu.sync_copy(ei_hbm.at[base + 1], ib)
        pltpu.sync_copy(ei_hbm.at[base + 2], ic)
        pltpu.async_copy(g_hbm.at[ia.at[0]], bufa, sema)
        pltpu.async_copy(g_hbm.at[ib.at[0]], bufb, semb)
        pltpu.async_copy(g_hbm.at[ic.at[0]], bufc, semc)

        def body(i, _):
            j = 3 * i
            pltpu.make_async_copy(g_hbm.at[ia.at[0]], bufa, sema).wait()
            pltpu.sync_copy(bufa, acc_sh.at[ia.at[1]], add=True)

            @pl.when(j + 3 < n_chunks)
            def _():
                pltpu.async_copy(ei_hbm.at[base + j + 3], ia, semia)

            @pl.when(j + 1 < n_chunks)
            def _():
                pltpu.make_async_copy(g_hbm.at[ib.at[0]], bufb, semb).wait()
                pltpu.sync_copy(bufb, acc_sh.at[ib.at[1]], add=True)

            @pl.when(j + 3 < n_chunks)
            def _():
                pltpu.make_async_copy(ei_hbm.at[base + j + 3], ia, semia).wait()
                pltpu.async_copy(g_hbm.at[ia.at[0]], bufa, sema)

            @pl.when(j + 4 < n_chunks)
            def _():
                pltpu.async_copy(ei_hbm.at[base + j + 4], ib, semib)

            @pl.when(j + 2 < n_chunks)
            def _():
                pltpu.make_async_copy(g_hbm.at[ic.at[0]], bufc, semc).wait()
                pltpu.sync_copy(bufc, acc_sh.at[ic.at[1]], add=True)

            @pl.when(j + 4 < n_chunks)
            def _():
                pltpu.make_async_copy(ei_hbm.at[base + j + 4], ib, semib).wait()
                pltpu.async_copy(g_hbm.at[ib.at[0]], bufb, semb)

            @pl.when(j + 5 < n_chunks)
            def _():
                pltpu.async_copy(ei_hbm.at[base + j + 5], ic, semic)
                pltpu.make_async_copy(ei_hbm.at[base + j + 5], ic, semic).wait()
                pltpu.async_copy(g_hbm.at[ic.at[0]], bufc, semc)
            return 0
        lax.fori_loop(0, (n_chunks + 2) // 3, body, 0)
        plsc.subcore_barrier()

        pltpu.sync_copy(acc_sh.at[pl.ds(s * rows_per_tile, rows_per_tile)],
                        out_hbm.at[c, pl.ds(s * rows_per_tile, rows_per_tile)])

    return scat_kernel


def _tc_scale_matmul(x, W, dinv):
    def body(x_ref, w_ref, d_ref, o_ref):
        h = jnp.dot(x_ref[...], w_ref[...], preferred_element_type=jnp.float32)
        o_ref[...] = h * d_ref[...]

    return pl.pallas_call(
        body,
        grid=(N_NODES // ROW_B,),
        in_specs=[
            pl.BlockSpec((ROW_B, D), lambda i: (i, 0)),
            pl.BlockSpec((D, D), lambda i: (0, 0)),
            pl.BlockSpec((ROW_B, 1), lambda i: (i, 0)),
        ],
        out_specs=pl.BlockSpec((ROW_B, D), lambda i: (i, 0)),
        out_shape=jax.ShapeDtypeStruct((N_NODES, D), jnp.float32),
    )(x, W, dinv)


def _tc_mid(parts, g, dinv, b, W):
    def body(p_ref, g_ref, d_ref, b_ref, w_ref, o_ref):
        agg = p_ref[0] + p_ref[1] + g_ref[...]
        t = jnp.maximum(agg * d_ref[...] + b_ref[...], 0.0)
        h = jnp.dot(t, w_ref[...], preferred_element_type=jnp.float32)
        o_ref[...] = h * d_ref[...]

    return pl.pallas_call(
        body,
        grid=(N_NODES // ROW_B,),
        in_specs=[
            pl.BlockSpec((NC, ROW_B, D), lambda i: (0, i, 0)),
            pl.BlockSpec((ROW_B, D), lambda i: (i, 0)),
            pl.BlockSpec((ROW_B, 1), lambda i: (i, 0)),
            pl.BlockSpec((1, D), lambda i: (0, 0)),
            pl.BlockSpec((D, D), lambda i: (0, 0)),
        ],
        out_specs=pl.BlockSpec((ROW_B, D), lambda i: (i, 0)),
        out_shape=jax.ShapeDtypeStruct((N_NODES, D), jnp.float32),
    )(parts, g, dinv, b, W)


def _tc_out(parts, g, dinv, b):
    def body(p_ref, g_ref, d_ref, b_ref, o_ref):
        agg = p_ref[0] + p_ref[1] + g_ref[...]
        z = agg * d_ref[...] + b_ref[...]
        m = jnp.max(z, axis=1, keepdims=True)
        e = jnp.exp(z - m)
        ssum = jnp.sum(e, axis=1, keepdims=True)
        o_ref[...] = (z - m) - jnp.log(ssum)

    return pl.pallas_call(
        body,
        grid=(N_NODES // ROW_B,),
        in_specs=[
            pl.BlockSpec((NC, ROW_B, D), lambda i: (0, i, 0)),
            pl.BlockSpec((ROW_B, D), lambda i: (i, 0)),
            pl.BlockSpec((ROW_B, 1), lambda i: (i, 0)),
            pl.BlockSpec((1, D), lambda i: (0, 0)),
        ],
        out_specs=pl.BlockSpec((ROW_B, D), lambda i: (i, 0)),
        out_shape=jax.ShapeDtypeStruct((N_NODES, D), jnp.float32),
    )(parts, g, dinv, b)


def kernel(x, edge_index, W1, b1, W2, b2):
    src = edge_index[0]
    dst = edge_index[1]
    e = src.shape[0]
    grain = NW * CHUNK
    e_pad = ((e + grain - 1) // grain) * grain
    n_chunks = e_pad // grain

    src_p = jnp.concatenate(
        [src, jnp.zeros((e_pad - e,), jnp.int32)]).reshape(NW, n_chunks, CHUNK)
    dst_p = jnp.concatenate(
        [dst, jnp.full((e_pad - e,), N_NODES, jnp.int32)]
    ).reshape(NW, n_chunks, CHUNK)
    t_chunks = e_pad // CHUNK
    ei_flat = jnp.stack(
        [src_p.reshape(t_chunks, CHUNK), dst_p.reshape(t_chunks, CHUNK)],
        axis=1)
    zz = jnp.zeros((N_ACC // NS, D), jnp.float32)

    deg_parts = _make_deg_kernel(n_chunks)(dst_p, zz)
    deg = deg_parts[:, :N_NODES, 0].sum(axis=0) + 1.0
    dinv = lax.rsqrt(deg).reshape(N_NODES, 1)

    k0 = int(round(CORE0_FRAC * t_chunks / NS))
    k1 = t_chunks // NS - k0
    scat = _make_scatter_kernel(k0, k1)
    b1r = b1.reshape(1, D)
    b2r = b2.reshape(1, D)

    g1 = _tc_scale_matmul(x, W1, dinv)
    parts1 = scat(g1, ei_flat, zz)[:, :N_NODES, :]
    g2 = _tc_mid(parts1, g1, dinv, b1r, W2)
    parts2 = scat(g2, ei_flat, zz)[:, :N_NODES, :]
    return _tc_out(parts2, g2, dinv, b2r)


# speedup vs baseline: 1.2298x; 1.0155x over previous
import functools

import jax
import jax.numpy as jnp
from jax import lax
from jax.experimental import pallas as pl
from jax.experimental.pallas import tpu as pltpu
from jax.experimental.pallas import tpu_sc as plsc

N_NODES = 10000
D = 128
NC = 2
NS = 16
LANES = 16
NW = NC * NS
CHUNK = 96
N_ACC = 10240
ROW_B = 1000
CORE0_FRAC = 0.78


def _sc_mesh():
    return plsc.VectorSubcoreMesh(
        core_axis_name="c", subcore_axis_name="s",
        num_cores=NC, num_subcores=NS)


def _make_deg_kernel(n_chunks):
    @functools.partial(
        pl.kernel,
        out_type=jax.ShapeDtypeStruct((NC, N_ACC, D), jnp.float32),
        mesh=_sc_mesh(),
        scratch_types=[
            pltpu.VMEM((n_chunks, CHUNK), jnp.int32),
            pltpu.VMEM((CHUNK, D), jnp.float32),
            pltpu.VMEM_SHARED((N_ACC, D), jnp.float32),
        ],
    )
    def deg_kernel(dst_hbm, zz_hbm, out_hbm, dst_v, ones_v, acc_sh):
        c = lax.axis_index("c")
        s = lax.axis_index("s")
        w = c * NS + s
        pltpu.sync_copy(dst_hbm.at[w], dst_v)

        rows_per_tile = N_ACC // NS
        onesvec = jnp.ones((LANES,), jnp.float32)
        per_row = D // LANES

        def fill(i, _):
            r = i // per_row
            k = i - r * per_row
            ones_v[r, pl.ds(k * LANES, LANES)] = onesvec
            return 0
        lax.fori_loop(0, CHUNK * per_row, fill, 0)

        pltpu.sync_copy(
            zz_hbm, acc_sh.at[pl.ds(s * rows_per_tile, rows_per_tile)])
        plsc.subcore_barrier()

        def body(j, _):
            pltpu.sync_copy(ones_v, acc_sh.at[dst_v.at[j]], add=True)
            return 0
        lax.fori_loop(0, n_chunks, body, 0)
        plsc.subcore_barrier()

        pltpu.sync_copy(acc_sh.at[pl.ds(s * rows_per_tile, rows_per_tile)],
                        out_hbm.at[c, pl.ds(s * rows_per_tile, rows_per_tile)])

    return deg_kernel


def _make_scatter_kernel(k0, k1):
    @functools.partial(
        pl.kernel,
        out_type=jax.ShapeDtypeStruct((NC, N_ACC, D), jnp.float32),
        mesh=_sc_mesh(),
        scratch_types=[
            pltpu.VMEM((2, CHUNK), jnp.int32),
            pltpu.VMEM((2, CHUNK), jnp.int32),
            pltpu.VMEM((2, CHUNK), jnp.int32),
            pltpu.VMEM((CHUNK, D), jnp.float32),
            pltpu.VMEM((CHUNK, D), jnp.float32),
            pltpu.VMEM((CHUNK, D), jnp.float32),
            pltpu.VMEM_SHARED((N_ACC, D), jnp.float32),
            pltpu.SemaphoreType.DMA,
            pltpu.SemaphoreType.DMA,
            pltpu.SemaphoreType.DMA,
            pltpu.SemaphoreType.DMA,
            pltpu.SemaphoreType.DMA,
            pltpu.SemaphoreType.DMA,
        ],
    )
    def scat_kernel(g_hbm, ei_hbm, zz_hbm, out_hbm,
                    ia, ib, ic, bufa, bufb, bufc, acc_sh,
                    sema, semb, semc, semia, semib, semic):
        c = lax.axis_index("c")
        s = lax.axis_index("s")
        base = jnp.where(c == 0, s * k0, NS * k0 + s * k1)
        n_chunks = jnp.where(c == 0, k0, k1)
        rows_per_tile = N_ACC // NS

        pltpu.sync_copy(
            zz_hbm, acc_sh.at[pl.ds(s * rows_per_tile, rows_per_tile)])
        plsc.subcore_barrier()

        pltpu.sync_copy(ei_hbm.at[base], ia)
        pltpu.sync_copy(ei_hbm.at[base + 1], ib)
        pltpu.sync_copy(ei_hbm.at[base + 2], ic)
        pltpu.async_copy(g_hbm.at[ia.at[0]], bufa, sema)
        pltpu.async_copy(g_hbm.at[ib.at[0]], bufb, semb)
        pltpu.async_copy(g_hbm.at[ic.at[0]], bufc, semc)

        def body(i, _):
            j = 3 * i
            pltpu.make_async_copy(g_hbm.at[ia.at[0]], bufa, sema).wait()
            pltpu.sync_copy(bufa, acc_sh.at[ia.at[1]], add=True)

            @pl.when(j + 3 < n_chunks)
            def _():
                pltpu.async_copy(ei_hbm.at[base + j + 3], ia, semia)

            @pl.when(j + 1 < n_chunks)
            def _():
                pltpu.make_async_copy(g_hbm.at[ib.at[0]], bufb, semb).wait()
                pltpu.sync_copy(bufb, acc_sh.at[ib.at[1]], add=True)

            @pl.when(j + 3 < n_chunks)
            def _():
                pltpu.make_async_copy(ei_hbm.at[base + j + 3], ia, semia).wait()
                pltpu.async_copy(g_hbm.at[ia.at[0]], bufa, sema)

            @pl.when(j + 4 < n_chunks)
            def _():
                pltpu.async_copy(ei_hbm.at[base + j + 4], ib, semib)

            @pl.when(j + 2 < n_chunks)
            def _():
                pltpu.make_async_copy(g_hbm.at[ic.at[0]], bufc, semc).wait()
                pltpu.sync_copy(bufc, acc_sh.at[ic.at[1]], add=True)

            @pl.when(j + 4 < n_chunks)
            def _():
                pltpu.make_async_copy(ei_hbm.at[base + j + 4], ib, semib).wait()
                pltpu.async_copy(g_hbm.at[ib.at[0]], bufb, semb)

            @pl.when(j + 5 < n_chunks)
            def _():
                pltpu.async_copy(ei_hbm.at[base + j + 5], ic, semic)
                pltpu.make_async_copy(ei_hbm.at[base + j + 5], ic, semic).wait()
                pltpu.async_copy(g_hbm.at[ic.at[0]], bufc, semc)
            return 0
        lax.fori_loop(0, (n_chunks + 2) // 3, body, 0)
        plsc.subcore_barrier()

        pltpu.sync_copy(acc_sh.at[pl.ds(s * rows_per_tile, rows_per_tile)],
                        out_hbm.at[c, pl.ds(s * rows_per_tile, rows_per_tile)])

    return scat_kernel


def _tc_scale_matmul(x, W, dinv):
    def body(x_ref, w_ref, d_ref, o_ref):
        h = jnp.dot(x_ref[...], w_ref[...], preferred_element_type=jnp.float32)
        o_ref[...] = h * d_ref[...]

    return pl.pallas_call(
        body,
        grid=(N_NODES // ROW_B,),
        in_specs=[
            pl.BlockSpec((ROW_B, D), lambda i: (i, 0)),
            pl.BlockSpec((D, D), lambda i: (0, 0)),
            pl.BlockSpec((ROW_B, 1), lambda i: (i, 0)),
        ],
        out_specs=pl.BlockSpec((ROW_B, D), lambda i: (i, 0)),
        out_shape=jax.ShapeDtypeStruct((N_NODES, D), jnp.float32),
    )(x, W, dinv)


def _tc_mid(parts, g, dinv, b, W):
    def body(p_ref, g_ref, d_ref, b_ref, w_ref, o_ref):
        agg = p_ref[0] + p_ref[1] + g_ref[...]
        t = jnp.maximum(agg * d_ref[...] + b_ref[...], 0.0)
        h = jnp.dot(t, w_ref[...], preferred_element_type=jnp.float32)
        o_ref[...] = h * d_ref[...]

    return pl.pallas_call(
        body,
        grid=(N_NODES // ROW_B,),
        in_specs=[
            pl.BlockSpec((NC, ROW_B, D), lambda i: (0, i, 0)),
            pl.BlockSpec((ROW_B, D), lambda i: (i, 0)),
            pl.BlockSpec((ROW_B, 1), lambda i: (i, 0)),
            pl.BlockSpec((1, D), lambda i: (0, 0)),
            pl.BlockSpec((D, D), lambda i: (0, 0)),
        ],
        out_specs=pl.BlockSpec((ROW_B, D), lambda i: (i, 0)),
        out_shape=jax.ShapeDtypeStruct((N_NODES, D), jnp.float32),
    )(parts, g, dinv, b, W)


def _tc_out(parts, g, dinv, b):
    def body(p_ref, g_ref, d_ref, b_ref, o_ref):
        agg = p_ref[0] + p_ref[1] + g_ref[...]
        z = agg * d_ref[...] + b_ref[...]
        m = jnp.max(z, axis=1, keepdims=True)
        e = jnp.exp(z - m)
        ssum = jnp.sum(e, axis=1, keepdims=True)
        o_ref[...] = (z - m) - jnp.log(ssum)

    return pl.pallas_call(
        body,
        grid=(N_NODES // ROW_B,),
        in_specs=[
            pl.BlockSpec((NC, ROW_B, D), lambda i: (0, i, 0)),
            pl.BlockSpec((ROW_B, D), lambda i: (i, 0)),
            pl.BlockSpec((ROW_B, 1), lambda i: (i, 0)),
            pl.BlockSpec((1, D), lambda i: (0, 0)),
        ],
        out_specs=pl.BlockSpec((ROW_B, D), lambda i: (i, 0)),
        out_shape=jax.ShapeDtypeStruct((N_NODES, D), jnp.float32),
    )(parts, g, dinv, b)


def kernel(x, edge_index, W1, b1, W2, b2):
    src = edge_index[0]
    dst = edge_index[1]
    e = src.shape[0]
    grain = NW * CHUNK
    e_pad = ((e + grain - 1) // grain) * grain
    n_chunks = e_pad // grain

    src_p = jnp.concatenate(
        [src, jnp.zeros((e_pad - e,), jnp.int32)]).reshape(NW, n_chunks, CHUNK)
    dst_p = jnp.concatenate(
        [dst, jnp.full((e_pad - e,), N_NODES, jnp.int32)]
    ).reshape(NW, n_chunks, CHUNK)
    t_chunks = e_pad // CHUNK
    ei_flat = jnp.stack(
        [src_p.reshape(t_chunks, CHUNK), dst_p.reshape(t_chunks, CHUNK)],
        axis=1)
    zz = jnp.zeros((N_ACC // NS, D), jnp.float32)

    deg_parts = _make_deg_kernel(n_chunks)(dst_p, zz)
    deg = deg_parts[:, :N_NODES, 0].sum(axis=0) + 1.0
    dinv = lax.rsqrt(deg).reshape(N_NODES, 1)

    k0 = int(round(CORE0_FRAC * t_chunks / NS))
    k1 = t_chunks // NS - k0
    scat = _make_scatter_kernel(k0, k1)
    b1r = b1.reshape(1, D)
    b2r = b2.reshape(1, D)

    g1 = _tc_scale_matmul(x, W1, dinv)
    parts1 = scat(g1, ei_flat, zz)[:, :N_NODES, :]
    g2 = _tc_mid(parts1, g1, dinv, b1r, W2)
    parts2 = scat(g2, ei_flat, zz)[:, :N_NODES, :]
    return _tc_out(parts2, g2, dinv, b2r)
